# baseline (speedup 1.0000x reference)
import functools

import jax
import jax.numpy as jnp
from jax import lax
from jax.experimental import pallas as pl
from jax.experimental.pallas import tpu as pltpu
from jax.experimental.pallas import tpu_sc as plsc

VOCAB_END = 50000
EMBED = 128
PAD_ROW = VOCAB_END

NC, NS = 2, 16
NW = NC * NS
CB = 512
G = CB // 128


def _emb_body(ids_hbm, table_hbm, out_hbm, idxv, rows, sem, *, rows_per_w):
    wid = lax.axis_index("s") * NC + lax.axis_index("c")
    row0 = wid * rows_per_w
    n_chunks = rows_per_w // G

    def chunk(ci, carry):
        rbase = row0 + ci * G
        pltpu.sync_copy(ids_hbm.at[pl.ds(rbase, G)], idxv)
        for g in range(G):
            for v in range(8):
                sl = (g, pl.ds(v * 16, 16))
                ids = idxv[sl]
                idxv[sl] = jnp.where(ids < VOCAB_END, ids, PAD_ROW)
        cps = [
            pltpu.async_copy(
                table_hbm.at[idxv.at[g]], rows.at[pl.ds(g * 128, 128)], sem
            )
            for g in range(G)
        ]
        for c in cps:
            c.wait()
        pltpu.sync_copy(rows, out_hbm.at[pl.ds(rbase * 128, CB)])
        return carry

    lax.fori_loop(0, n_chunks, chunk, 0)


def _make_emb(b_tokens):
    assert b_tokens % (128 * G * NW) == 0
    rows_per_w = (b_tokens // 128) // NW
    mesh = plsc.VectorSubcoreMesh(core_axis_name="c", subcore_axis_name="s")
    return functools.partial(
        pl.kernel,
        mesh=mesh,
        out_type=jax.ShapeDtypeStruct((b_tokens, EMBED), jnp.float32),
        scratch_types=[
            pltpu.VMEM((G, 128), jnp.int32),
            pltpu.VMEM((CB, EMBED), jnp.float32),
            pltpu.SemaphoreType.DMA,
        ],
    )(functools.partial(_emb_body, rows_per_w=rows_per_w))


def kernel(input_ids, weight):
    b, t = input_ids.shape
    ids = input_ids.reshape(-1).astype(jnp.int32).reshape(-1, 128)
    wpad = jnp.pad(weight, ((0, 8), (0, 0)))
    out = _make_emb(b * t)(ids, wpad)
    return out.reshape(b, t, EMBED)


# speedup vs baseline: 1.0004x; 1.0004x over previous
import functools

import jax
import jax.numpy as jnp
from jax import lax
from jax.experimental import pallas as pl
from jax.experimental.pallas import tpu as pltpu
from jax.experimental.pallas import tpu_sc as plsc

VOCAB_END = 50000
EMBED = 128
PAD_ROW = VOCAB_END

NC, NS = 2, 16
NW = NC * NS
CB = 128
NB = 5


def _stage_and_fire(ids_hbm, table_hbm, idxv, rows, gsem, b, rowb):
    pltpu.sync_copy(ids_hbm.at[pl.ds(rowb, 1)], idxv.at[pl.ds(b, 1)])
    for v in range(8):
        sl = (b, pl.ds(v * 16, 16))
        ids = idxv[sl]
        idxv[sl] = jnp.where(ids < VOCAB_END, ids, PAD_ROW)
    return pltpu.async_copy(table_hbm.at[idxv.at[b]], rows.at[b], gsem[b])


def _emb_body(ids_hbm, table_hbm, out_hbm, idxv, rows, *sems, rows_per_w):
    gsem, osem = sems[:NB], sems[NB:]
    wid = lax.axis_index("s") * NC + lax.axis_index("c")
    row0 = wid * rows_per_w
    n_chunks = rows_per_w

    for b in range(NB):
        _stage_and_fire(ids_hbm, table_hbm, idxv, rows, gsem, b, row0 + b)

    def ring_cycle(ci, carry):
        base = row0 + ci * NB
        for b in range(NB):
            pltpu.make_async_copy(
                table_hbm.at[idxv.at[b]], rows.at[b], gsem[b]
            ).wait()
            ocp = pltpu.async_copy(
                rows.at[b], out_hbm.at[pl.ds((base + b) * CB, CB)], osem[b]
            )
            nxt = base + NB + b
            pltpu.sync_copy(ids_hbm.at[pl.ds(nxt, 1)], idxv.at[pl.ds(b, 1)])
            for v in range(8):
                sl = (b, pl.ds(v * 16, 16))
                ids = idxv[sl]
                idxv[sl] = jnp.where(ids < VOCAB_END, ids, PAD_ROW)
            ocp.wait()
            pltpu.async_copy(table_hbm.at[idxv.at[b]], rows.at[b], gsem[b])
        return carry

    lax.fori_loop(0, n_chunks // NB - 1, ring_cycle, 0)

    base = row0 + n_chunks - NB
    for b in range(NB):
        pltpu.make_async_copy(
            table_hbm.at[idxv.at[b]], rows.at[b], gsem[b]
        ).wait()
        pltpu.sync_copy(rows.at[b], out_hbm.at[pl.ds((base + b) * CB, CB)])


def _make_emb(b_tokens):
    assert b_tokens % (CB * NB * NW) == 0
    rows_per_w = (b_tokens // CB) // NW
    mesh = plsc.VectorSubcoreMesh(core_axis_name="c", subcore_axis_name="s")
    return functools.partial(
        pl.kernel,
        mesh=mesh,
        out_type=jax.ShapeDtypeStruct((b_tokens, EMBED), jnp.float32),
        scratch_types=[
            pltpu.VMEM((NB, CB), jnp.int32),
            pltpu.VMEM((NB, CB, EMBED), jnp.float32),
        ]
        + [pltpu.SemaphoreType.DMA] * (2 * NB),
    )(functools.partial(_emb_body, rows_per_w=rows_per_w))


def kernel(input_ids, weight):
    b, t = input_ids.shape
    ids = input_ids.reshape(-1).astype(jnp.int32).reshape(-1, CB)
    wpad = jnp.pad(weight, ((0, 8), (0, 0)))
    out = _make_emb(b * t)(ids, wpad)
    return out.reshape(b, t, EMBED)


# speedup vs baseline: 12.0128x; 12.0075x over previous
import functools

import jax
import jax.numpy as jnp
from jax import lax
from jax.experimental import pallas as pl
from jax.experimental.pallas import tpu as pltpu

VOCAB_END = 50000
EMBED = 128
PAD_ROW = VOCAB_END

BT = 1024


def _tc_body(ids_ref, table_ref, out_ref):
    def tok(j, carry):
        i = jnp.minimum(ids_ref[j], PAD_ROW)
        out_ref[pl.ds(j, 1), :] = table_ref[pl.ds(i, 1), :]
        return carry

    lax.fori_loop(0, BT, tok, 0, unroll=16)


def _make_tc(b_tokens):
    assert b_tokens % BT == 0
    return pl.pallas_call(
        _tc_body,
        grid=(b_tokens // BT,),
        in_specs=[
            pl.BlockSpec((BT,), lambda i: (i,), memory_space=pltpu.SMEM),
            pl.BlockSpec((VOCAB_END + 8, EMBED), lambda i: (0, 0)),
        ],
        out_specs=pl.BlockSpec((BT, EMBED), lambda i: (i, 0)),
        out_shape=jax.ShapeDtypeStruct((b_tokens, EMBED), jnp.float32),
        compiler_params=pltpu.CompilerParams(
            dimension_semantics=("arbitrary",),
        ),
    )


def kernel(input_ids, weight):
    b, t = input_ids.shape
    ids = input_ids.reshape(-1).astype(jnp.int32)
    wpad = jnp.pad(weight, ((0, 8), (0, 0)))
    out = _make_tc(b * t)(ids, wpad)
    return out.reshape(b, t, EMBED)


# speedup vs baseline: 13.0312x; 1.0848x over previous
import functools

import jax
import jax.numpy as jnp
from jax import lax
from jax.experimental import pallas as pl
from jax.experimental.pallas import tpu as pltpu

VOCAB_END = 50000
EMBED = 128
PAD_ROW = VOCAB_END

BT = 2048


def _tc_body(ids_ref, table_ref, out_ref):
    def tok(j, carry):
        i = jnp.minimum(ids_ref[j], PAD_ROW)
        out_ref[pl.ds(j, 1), :] = table_ref[pl.ds(i, 1), :]
        return carry

    lax.fori_loop(0, BT, tok, 0, unroll=32)


def _make_tc(b_tokens):
    assert b_tokens % BT == 0
    return pl.pallas_call(
        _tc_body,
        grid=(b_tokens // BT,),
        in_specs=[
            pl.BlockSpec((BT,), lambda i: (i,), memory_space=pltpu.SMEM),
            pl.BlockSpec((VOCAB_END + 8, EMBED), lambda i: (0, 0)),
        ],
        out_specs=pl.BlockSpec((BT, EMBED), lambda i: (i, 0)),
        out_shape=jax.ShapeDtypeStruct((b_tokens, EMBED), jnp.float32),
        compiler_params=pltpu.CompilerParams(
            dimension_semantics=("arbitrary",),
        ),
    )


def kernel(input_ids, weight):
    b, t = input_ids.shape
    ids = input_ids.reshape(-1).astype(jnp.int32)
    wpad = jnp.pad(weight, ((0, 8), (0, 0)))
    out = _make_tc(b * t)(ids, wpad)
    return out.reshape(b, t, EMBED)


# speedup vs baseline: 19.9792x; 1.5332x over previous
import functools

import jax
import jax.numpy as jnp
from jax import lax
from jax.experimental import pallas as pl
from jax.experimental.pallas import tpu as pltpu

VOCAB_END = 50000
FULL_ROWS = 100008
EMBED = 128

BT = 2048


def _tc_body(ids_ref, table_ref, out_ref):
    def tok(j, carry):
        out_ref[pl.ds(j, 1), :] = table_ref[pl.ds(ids_ref[j], 1), :]
        return carry

    lax.fori_loop(0, BT, tok, 0, unroll=32)


def _make_tc(b_tokens):
    assert b_tokens % BT == 0
    return pl.pallas_call(
        _tc_body,
        grid=(b_tokens // BT,),
        in_specs=[
            pl.BlockSpec((BT,), lambda i: (i,), memory_space=pltpu.SMEM),
            pl.BlockSpec((FULL_ROWS, EMBED), lambda i: (0, 0)),
        ],
        out_specs=pl.BlockSpec((BT, EMBED), lambda i: (i, 0)),
        out_shape=jax.ShapeDtypeStruct((b_tokens, EMBED), jnp.float32),
        compiler_params=pltpu.CompilerParams(
            dimension_semantics=("arbitrary",),
        ),
    )


def kernel(input_ids, weight):
    b, t = input_ids.shape
    ids = input_ids.reshape(-1).astype(jnp.int32)
    wpad = jnp.pad(weight, ((0, FULL_ROWS - VOCAB_END), (0, 0)))
    out = _make_tc(b * t)(ids, wpad)
    return out.reshape(b, t, EMBED)


# speedup vs baseline: 21.7132x; 1.0868x over previous
import functools

import jax
import jax.numpy as jnp
from jax import lax
from jax.experimental import pallas as pl
from jax.experimental.pallas import tpu as pltpu

VOCAB_END = 50000
FULL_ROWS = 100008
EMBED = 128

BT = 4096


def _tc_body(ids_ref, table_ref, out_ref):
    def tok(j, carry):
        out_ref[pl.ds(j, 1), :] = table_ref[pl.ds(ids_ref[j], 1), :]
        return carry

    lax.fori_loop(0, BT, tok, 0, unroll=64)


def _make_tc(b_tokens):
    assert b_tokens % BT == 0
    return pl.pallas_call(
        _tc_body,
        grid=(b_tokens // BT,),
        in_specs=[
            pl.BlockSpec((BT,), lambda i: (i,), memory_space=pltpu.SMEM),
            pl.BlockSpec((FULL_ROWS, EMBED), lambda i: (0, 0)),
        ],
        out_specs=pl.BlockSpec((BT, EMBED), lambda i: (i, 0)),
        out_shape=jax.ShapeDtypeStruct((b_tokens, EMBED), jnp.float32),
        compiler_params=pltpu.CompilerParams(
            dimension_semantics=("arbitrary",),
        ),
    )


def kernel(input_ids, weight):
    b, t = input_ids.shape
    ids = input_ids.reshape(-1).astype(jnp.int32)
    wpad = jnp.pad(weight, ((0, FULL_ROWS - VOCAB_END), (0, 0)))
    out = _make_tc(b * t)(ids, wpad)
    return out.reshape(b, t, EMBED)


# speedup vs baseline: 22.6006x; 1.0409x over previous
import functools

import jax
import jax.numpy as jnp
from jax import lax
from jax.experimental import pallas as pl
from jax.experimental.pallas import tpu as pltpu

VOCAB_END = 50000
FULL_ROWS = 100008
EMBED = 128

BT = 4096


def _tc_body(ids_ref, table_ref, out_ref):
    def tok(j, carry):
        out_ref[pl.ds(j, 1), :] = table_ref[pl.ds(ids_ref[j], 1), :]
        return carry

    lax.fori_loop(0, BT, tok, 0, unroll=128)


def _make_tc(b_tokens):
    assert b_tokens % BT == 0
    return pl.pallas_call(
        _tc_body,
        grid=(b_tokens // BT,),
        in_specs=[
            pl.BlockSpec((BT,), lambda i: (i,), memory_space=pltpu.SMEM),
            pl.BlockSpec((FULL_ROWS, EMBED), lambda i: (0, 0)),
        ],
        out_specs=pl.BlockSpec((BT, EMBED), lambda i: (i, 0)),
        out_shape=jax.ShapeDtypeStruct((b_tokens, EMBED), jnp.float32),
        compiler_params=pltpu.CompilerParams(
            dimension_semantics=("arbitrary",),
        ),
    )


def kernel(input_ids, weight):
    b, t = input_ids.shape
    ids = input_ids.reshape(-1).astype(jnp.int32)
    wpad = jnp.pad(weight, ((0, FULL_ROWS - VOCAB_END), (0, 0)))
    out = _make_tc(b * t)(ids, wpad)
    return out.reshape(b, t, EMBED)


# speedup vs baseline: 23.0704x; 1.0208x over previous
import functools

import jax
import jax.numpy as jnp
from jax import lax
from jax.experimental import pallas as pl
from jax.experimental.pallas import tpu as pltpu

VOCAB_END = 50000
FULL_ROWS = 100008
EMBED = 128

BT = 4096


def _tc_body(ids_ref, table_ref, out_ref):
    def tok(j, carry):
        out_ref[pl.ds(j, 1), :] = table_ref[pl.ds(ids_ref[j], 1), :]
        return carry

    lax.fori_loop(0, BT, tok, 0, unroll=256)


def _make_tc(b_tokens):
    assert b_tokens % BT == 0
    return pl.pallas_call(
        _tc_body,
        grid=(b_tokens // BT,),
        in_specs=[
            pl.BlockSpec((BT,), lambda i: (i,), memory_space=pltpu.SMEM),
            pl.BlockSpec((FULL_ROWS, EMBED), lambda i: (0, 0)),
        ],
        out_specs=pl.BlockSpec((BT, EMBED), lambda i: (i, 0)),
        out_shape=jax.ShapeDtypeStruct((b_tokens, EMBED), jnp.float32),
        compiler_params=pltpu.CompilerParams(
            dimension_semantics=("arbitrary",),
        ),
    )


def kernel(input_ids, weight):
    b, t = input_ids.shape
    ids = input_ids.reshape(-1).astype(jnp.int32)
    wpad = jnp.pad(weight, ((0, FULL_ROWS - VOCAB_END), (0, 0)))
    out = _make_tc(b * t)(ids, wpad)
    return out.reshape(b, t, EMBED)


# speedup vs baseline: 23.3024x; 1.0101x over previous
import functools

import jax
import jax.numpy as jnp
from jax import lax
from jax.experimental import pallas as pl
from jax.experimental.pallas import tpu as pltpu

VOCAB_END = 50000
FULL_ROWS = 100008
EMBED = 128

BT = 4096


def _tc_body(ids_ref, table_ref, out_ref):
    def tok(j, carry):
        out_ref[pl.ds(j, 1), :] = table_ref[pl.ds(ids_ref[j], 1), :]
        return carry

    lax.fori_loop(0, BT, tok, 0, unroll=512)


def _make_tc(b_tokens):
    assert b_tokens % BT == 0
    return pl.pallas_call(
        _tc_body,
        grid=(b_tokens // BT,),
        in_specs=[
            pl.BlockSpec((BT,), lambda i: (i,), memory_space=pltpu.SMEM),
            pl.BlockSpec((FULL_ROWS, EMBED), lambda i: (0, 0)),
        ],
        out_specs=pl.BlockSpec((BT, EMBED), lambda i: (i, 0)),
        out_shape=jax.ShapeDtypeStruct((b_tokens, EMBED), jnp.float32),
        compiler_params=pltpu.CompilerParams(
            dimension_semantics=("arbitrary",),
        ),
    )


def kernel(input_ids, weight):
    b, t = input_ids.shape
    ids = input_ids.reshape(-1).astype(jnp.int32)
    wpad = jnp.pad(weight, ((0, FULL_ROWS - VOCAB_END), (0, 0)))
    out = _make_tc(b * t)(ids, wpad)
    return out.reshape(b, t, EMBED)


# speedup vs baseline: 23.3239x; 1.0009x over previous
import jax
import jax.numpy as jnp
from jax import lax
from jax.experimental import pallas as pl
from jax.experimental.pallas import tpu as pltpu

VOCAB_END = 50000
FULL_ROWS = 100008
EMBED = 128

BT = 4096


def _tc_body(ids_ref, table_ref, out_ref):
    def tok(j, carry):
        out_ref[pl.ds(j, 1), :] = table_ref[pl.ds(ids_ref[j], 1), :]
        return carry

    lax.fori_loop(0, BT, tok, 0, unroll=512)


def _make_tc(b_tokens):
    assert b_tokens % BT == 0
    return pl.pallas_call(
        _tc_body,
        grid=(b_tokens // BT,),
        in_specs=[
            pl.BlockSpec((BT,), lambda i: (i,), memory_space=pltpu.SMEM),
            pl.BlockSpec((FULL_ROWS, EMBED), lambda i: (0, 0)),
        ],
        out_specs=pl.BlockSpec((BT, EMBED), lambda i: (i, 0)),
        out_shape=jax.ShapeDtypeStruct((b_tokens, EMBED), jnp.float32),
        compiler_params=pltpu.CompilerParams(
            dimension_semantics=("arbitrary",),
        ),
    )


def kernel(input_ids, weight):
    b, t = input_ids.shape
    ids = input_ids.reshape(-1).astype(jnp.int32)
    wpad = jnp.pad(weight, ((0, FULL_ROWS - VOCAB_END), (0, 0)))
    out = _make_tc(b * t)(ids, wpad)
    return out.reshape(b, t, EMBED)
